# baseline (speedup 1.0000x reference)
import functools
import math

import jax
import jax.numpy as jnp
from jax import lax
from jax.experimental import pallas as pl
from jax.experimental.pallas import tpu as pltpu
from jax.experimental.pallas import tpu_sc as plsc

N_TOK = 1000000
D = 64
SCALE = math.sqrt(D)

NC = 2
NS = 16
NW = NC * NS
L = 16

CHUNK = 128
NBUF = 2


def _emb_kernel(n_chunks, idx_hbm, lut_hbm, out_hbm,
                idx_v, rows_v, gsems, wsems):
    wid = lax.axis_index("s") * NC + lax.axis_index("c")
    pltpu.sync_copy(idx_hbm.at[wid], idx_v)

    def start_gather(j, b):
        pltpu.async_copy(lut_hbm.at[idx_v.at[j]], rows_v.at[b], gsems.at[b])

    for b in range(NBUF):
        start_gather(b, b)

    def body(j):
        for b in range(NBUF):
            jj = j + b
            pltpu.make_async_copy(lut_hbm.at[idx_v.at[jj]],
                                  rows_v.at[b], gsems.at[b]).wait()
            def scale(i):
                for d in range(D // L):
                    sl = (b, i, pl.ds(d * L, L))
                    rows_v[sl] = rows_v[sl] * SCALE
            pl.loop(0, CHUNK)(scale)
            row0 = (wid * n_chunks + jj) * CHUNK
            pltpu.async_copy(rows_v.at[b],
                             out_hbm.at[pl.ds(row0, CHUNK)], wsems.at[b])
            @pl.when(jj + NBUF < n_chunks)
            def _():
                pltpu.make_async_copy(
                    rows_v.at[b],
                    out_hbm.at[pl.ds(row0, CHUNK)], wsems.at[b]).wait()
                start_gather(jj + NBUF, b)

    pl.loop(0, n_chunks, step=NBUF)(body)

    for b in range(NBUF):
        jj = n_chunks - NBUF + b
        row0 = (wid * n_chunks + jj) * CHUNK
        pltpu.make_async_copy(rows_v.at[b],
                              out_hbm.at[pl.ds(row0, CHUNK)], wsems.at[b]).wait()


@jax.jit
def kernel(x, lut):
    B = x.shape[0] * x.shape[1]
    n_chunks = B // (NW * CHUNK)
    idx = x.reshape(NW, n_chunks, CHUNK).astype(jnp.int32)

    mesh = plsc.VectorSubcoreMesh(core_axis_name="c", subcore_axis_name="s")
    run = pl.kernel(
        functools.partial(_emb_kernel, n_chunks),
        out_type=jax.ShapeDtypeStruct((B, D), jnp.float32),
        mesh=mesh,
        scratch_types=[
            pltpu.VMEM((n_chunks, CHUNK), jnp.int32),
            pltpu.VMEM((NBUF, CHUNK, D), jnp.float32),
            pltpu.SemaphoreType.DMA((NBUF,)),
            pltpu.SemaphoreType.DMA((NBUF,)),
        ],
        compiler_params=pltpu.CompilerParams(use_tc_tiling_on_sc=False),
    )
    out = run(idx, lut)
    return out.reshape(x.shape[0], x.shape[1], D)


# speedup vs baseline: 1.0601x; 1.0601x over previous
import functools
import math

import jax
import jax.numpy as jnp
from jax import lax
from jax.experimental import pallas as pl
from jax.experimental.pallas import tpu as pltpu
from jax.experimental.pallas import tpu_sc as plsc

D = 64
SCALE = math.sqrt(D)

NC = 2
NS = 16
NW = NC * NS
L = 16

CHUNK = 128
NBUF = 2
RUNROLL = 4


def _emb_kernel(n_chunks, idx_hbm, lut_hbm, out_hbm,
                idx_v, rows_v, gsems, wsems):
    wid = lax.axis_index("s") * NC + lax.axis_index("c")
    pltpu.sync_copy(idx_hbm.at[wid], idx_v)

    def start_gather(j, b):
        pltpu.async_copy(lut_hbm.at[idx_v.at[j]], rows_v.at[b], gsems.at[b])

    for b in range(NBUF):
        start_gather(b, b)

    def body(j):
        for b in range(NBUF):
            jj = j + b
            pltpu.make_async_copy(lut_hbm.at[idx_v.at[jj]],
                                  rows_v.at[b], gsems.at[b]).wait()
            def scale(i):
                for r in range(RUNROLL):
                    for d in range(D // L):
                        sl = (b, i + r, pl.ds(d * L, L))
                        rows_v[sl] = rows_v[sl] * SCALE
            pl.loop(0, CHUNK, step=RUNROLL)(scale)
            row0 = (wid * n_chunks + jj) * CHUNK
            pltpu.async_copy(rows_v.at[b],
                             out_hbm.at[pl.ds(row0, CHUNK)], wsems.at[b])
            @pl.when(jj + NBUF < n_chunks)
            def _():
                pltpu.make_async_copy(
                    rows_v.at[b],
                    out_hbm.at[pl.ds(row0, CHUNK)], wsems.at[b]).wait()
                start_gather(jj + NBUF, b)

    pl.loop(0, n_chunks, step=NBUF)(body)

    for b in range(NBUF):
        jj = n_chunks - NBUF + b
        row0 = (wid * n_chunks + jj) * CHUNK
        pltpu.make_async_copy(rows_v.at[b],
                              out_hbm.at[pl.ds(row0, CHUNK)], wsems.at[b]).wait()


@jax.jit
def kernel(x, lut):
    n_batch, n_pos = x.shape
    B = n_batch * n_pos
    n_chunks = B // (NW * CHUNK)
    idx = x.T.astype(jnp.int32).reshape(NW, n_chunks, CHUNK)

    mesh = plsc.VectorSubcoreMesh(core_axis_name="c", subcore_axis_name="s")
    run = pl.kernel(
        functools.partial(_emb_kernel, n_chunks),
        out_type=jax.ShapeDtypeStruct((B, D), jnp.float32),
        mesh=mesh,
        scratch_types=[
            pltpu.VMEM((n_chunks, CHUNK), jnp.int32),
            pltpu.VMEM((NBUF, CHUNK, D), jnp.float32),
            pltpu.SemaphoreType.DMA((NBUF,)),
            pltpu.SemaphoreType.DMA((NBUF,)),
        ],
        compiler_params=pltpu.CompilerParams(use_tc_tiling_on_sc=False),
    )
    out = run(idx, lut)
    return out.reshape(n_pos, n_batch, D).transpose(1, 0, 2)


# speedup vs baseline: 1.0621x; 1.0018x over previous
import functools
import math

import jax
import jax.numpy as jnp
from jax import lax
from jax.experimental import pallas as pl
from jax.experimental.pallas import tpu as pltpu
from jax.experimental.pallas import tpu_sc as plsc

D = 64
SCALE = math.sqrt(D)
N_PAIRS = 500000

NC = 2
NS = 16
NW = NC * NS
L = 16

CHUNK = 128
NBUF = 2
RUNROLL = 4


def _emb_kernel(n_chunks, idx_hbm, lut_hbm, out_hbm,
                idx_v, rows_v, gsems, wsems):
    wid = lax.axis_index("s") * NC + lax.axis_index("c")
    pltpu.sync_copy(idx_hbm.at[wid], idx_v)

    def start_gather(j, b):
        pltpu.async_copy(lut_hbm.at[idx_v.at[j]], rows_v.at[b], gsems.at[b])

    for b in range(NBUF):
        start_gather(b, b)

    def body(j):
        for b in range(NBUF):
            jj = j + b
            pltpu.make_async_copy(lut_hbm.at[idx_v.at[jj]],
                                  rows_v.at[b], gsems.at[b]).wait()
            def scale(i):
                for r in range(RUNROLL):
                    for d in range(D // L):
                        sl = (b, i + r, pl.ds(d * L, L))
                        rows_v[sl] = rows_v[sl] * SCALE
            pl.loop(0, CHUNK, step=RUNROLL)(scale)
            row0 = (wid * n_chunks + jj) * CHUNK
            pltpu.async_copy(rows_v.at[b],
                             out_hbm.at[pl.ds(row0, CHUNK)], wsems.at[b])
            @pl.when(jj + NBUF < n_chunks)
            def _():
                pltpu.make_async_copy(
                    rows_v.at[b],
                    out_hbm.at[pl.ds(row0, CHUNK)], wsems.at[b]).wait()
                start_gather(jj + NBUF, b)

    pl.loop(0, n_chunks, step=NBUF)(body)

    for b in range(NBUF):
        jj = n_chunks - NBUF + b
        row0 = (wid * n_chunks + jj) * CHUNK
        pltpu.make_async_copy(rows_v.at[b],
                              out_hbm.at[pl.ds(row0, CHUNK)], wsems.at[b]).wait()


@jax.jit
def kernel(x, lut):
    n_batch, n_pos = x.shape
    B = n_batch * n_pos
    n_chunks = B // (NW * CHUNK)
    idx = x.T.astype(jnp.int32).reshape(NW, n_chunks, CHUNK)
    lut2 = jax.lax.optimization_barrier(lut.reshape(N_PAIRS, 2 * D))
    lut_rm = lut2.reshape(lut.shape)

    mesh = plsc.VectorSubcoreMesh(core_axis_name="c", subcore_axis_name="s")
    run = pl.kernel(
        functools.partial(_emb_kernel, n_chunks),
        out_type=jax.ShapeDtypeStruct((B, D), jnp.float32),
        mesh=mesh,
        scratch_types=[
            pltpu.VMEM((n_chunks, CHUNK), jnp.int32),
            pltpu.VMEM((NBUF, CHUNK, D), jnp.float32),
            pltpu.SemaphoreType.DMA((NBUF,)),
            pltpu.SemaphoreType.DMA((NBUF,)),
        ],
        compiler_params=pltpu.CompilerParams(use_tc_tiling_on_sc=False),
    )
    out = run(idx, lut_rm)
    return out.reshape(n_pos, n_batch, D).transpose(1, 0, 2)
